# baseline (speedup 1.0000x reference)
import functools

import jax
import jax.numpy as jnp
from jax import lax
from jax.experimental import pallas as pl
from jax.experimental.pallas import tpu as pltpu
from jax.experimental.pallas import tpu_sc as plsc

N_NODES = 10000
N_PAD = 10240
N_EDGES = 320000
D = 128
NC = 2
NS = 16
L = 16
NW = NC * NS
EPW = N_EDGES // NW
K = 80
NCHUNK = EPW // K
NB = 5
NGRP = NCHUNK // NB
NPT = N_PAD // NS
RPT = N_PAD // NS
K2 = 40
NCHUNK2 = EPW // K2
NGRP2 = NCHUNK2 // NB


def _worker(cid, sid):
    return cid * NS + sid


def _deg_body(dst3_hbm, out_hbm, didx_v, ones_v, stage_v, rep_v, dsem,
              acc_sh):
    cid = lax.axis_index("c")
    sid = lax.axis_index("s")
    w = _worker(cid, sid)

    for i in range(K // L):
        ones_v[pl.ds(i * L, L)] = jnp.ones((L,), jnp.float32)

    def zbody(i, _):
        stage_v[pl.ds(i * L, L)] = jnp.zeros((L,), jnp.float32)
        return _

    lax.fori_loop(0, NPT // L, zbody, None)
    pltpu.sync_copy(stage_v, acc_sh.at[pl.ds(sid * NPT, NPT)])
    pltpu.sync_copy(dst3_hbm.at[w], didx_v)
    plsc.subcore_barrier()

    def sdesc(c):
        return pltpu.make_async_copy(ones_v, acc_sh.at[didx_v.at[c]], dsem)

    def gbody(g, _):
        for b in range(NB):
            sdesc(g * NB + b).start(add=True)
        for b in range(NB):
            sdesc(g * NB + b).wait()
        return _

    lax.fori_loop(0, NGRP, gbody, None)
    plsc.subcore_barrier()

    pltpu.sync_copy(acc_sh.at[pl.ds(sid * NPT, NPT)], stage_v)

    def rbody(i, _):
        rep_v[i, :] = plsc.load_gather(stage_v, [jnp.full((L,), i, jnp.int32)])
        return _

    lax.fori_loop(0, NPT, rbody, None)
    pltpu.sync_copy(rep_v, out_hbm.at[cid, pl.ds(sid * NPT, NPT), :])


def _spass_body(xs_hbm, src_hbm, dst_hbm, out_hbm,
                sidx_v, didx_v, rows_v,
                gs0, gs1, gs2, gs3, gs4, ss0, ss1, ss2, ss3, ss4,
                is0, is1, is2, is3, is4, js0, js1, js2, js3, js4, acc_sh):
    cid = lax.axis_index("c")
    sid = lax.axis_index("s")
    w = _worker(cid, sid)
    gsems = [gs0, gs1, gs2, gs3, gs4]
    ssems = [ss0, ss1, ss2, ss3, ss4]
    isems = [is0, is1, is2, is3, is4]
    jsems = [js0, js1, js2, js3, js4]

    def zbody(i, _):
        for j in range(D // L):
            rows_v[i, pl.ds(j * L, L)] = jnp.zeros((L,), jnp.float32)
        return _

    lax.fori_loop(0, NB * K2, zbody, None)
    zrows = NB * K2
    done = 0
    while done < RPT:
        n = min(zrows, RPT - done)
        pltpu.sync_copy(rows_v.at[pl.ds(0, n)],
                        acc_sh.at[pl.ds(sid * RPT + done, n), :])
        done += n

    def sidesc(c, b):
        base = pl.multiple_of(w * EPW + c * K2, 8)
        return pltpu.make_async_copy(src_hbm.at[pl.ds(base, K2)],
                                     sidx_v.at[b], isems[b])

    def didesc(c, b):
        base = pl.multiple_of(w * EPW + c * K2, 8)
        return pltpu.make_async_copy(dst_hbm.at[pl.ds(base, K2)],
                                     didx_v.at[b], jsems[b])

    def gdesc(c, b):
        del c
        return pltpu.make_async_copy(
            xs_hbm.at[sidx_v.at[b]], rows_v.at[pl.ds(b * K2, K2)], gsems[b])

    def sdesc(c, b):
        del c
        return pltpu.make_async_copy(
            rows_v.at[pl.ds(b * K2, K2)], acc_sh.at[didx_v.at[b]], ssems[b])

    for b in range(NB):
        sidesc(b, b).start()
        sidesc(b, b).wait()
        gdesc(b, b).start()
        didesc(b, b).start()
    plsc.subcore_barrier()

    def gbody(g, _):
        for b in range(NB):
            c = g * NB + b
            gdesc(c, b).wait()
            didesc(c, b).wait()
            sdesc(c, b).start(add=True)

            @pl.when(g < NGRP2 - 1)
            def _stage_sidx():
                sidesc(c + NB, b).start()

        @pl.when(g < NGRP2 - 1)
        def _refill():
            for b in range(NB):
                c = g * NB + b
                sdesc(c, b).wait()
                didesc(c + NB, b).start()
                sidesc(c + NB, b).wait()
                gdesc(c + NB, b).start()

        return _

    lax.fori_loop(0, NGRP2, gbody, None)
    for b in range(NB):
        sdesc((NGRP2 - 1) * NB + b, b).wait()
    plsc.subcore_barrier()

    pltpu.sync_copy(acc_sh.at[pl.ds(sid * RPT, RPT), :],
                    out_hbm.at[cid, pl.ds(sid * RPT, RPT), :])


def _score_body(h_hbm, src_hbm, dst_hbm, out_hbm,
                sidx_v, didx_v, hs_v, hd_v, pr_v, sc_v,
                s0, s1, s2, s3, s4):
    cid = lax.axis_index("c")
    sid = lax.axis_index("s")
    w = _worker(cid, sid)
    sems = [s0, s1, s2, s3, s4]
    iota = lax.iota(jnp.int32, L)

    pltpu.sync_copy(src_hbm.at[pl.ds(w * EPW, EPW)], sidx_v)
    pltpu.sync_copy(dst_hbm.at[pl.ds(w * EPW, EPW)], didx_v)

    def adesc(c, b):
        off = pl.multiple_of(c * K, 8)
        return pltpu.make_async_copy(
            h_hbm.at[sidx_v.at[pl.ds(off, K)]], hs_v.at[b], sems[b])

    def bdesc(c, b):
        off = pl.multiple_of(c * K, 8)
        return pltpu.make_async_copy(
            h_hbm.at[didx_v.at[pl.ds(off, K)]], hd_v.at[b], sems[b])

    for b in range(NB):
        adesc(b, b).start()
        bdesc(b, b).start()

    def gbody(g, _):
        for b in range(NB):
            c = g * NB + b
            adesc(c, b).wait()
            bdesc(c, b).wait()

            @plsc.parallel_loop(0, K, unroll=8)
            def _edot(e):
                acc = jnp.zeros((L,), jnp.float32)
                for k in range(D // L):
                    acc = acc + (hs_v[b, e, pl.ds(k * L, L)] *
                                 hd_v[b, e, pl.ds(k * L, L)])
                pr_v[pl.ds(pl.multiple_of(e * L, 8), L)] = acc

            for q in range(K // L):
                flat = (q * L + iota) * L
                tot = jnp.zeros((L,), jnp.float32)
                for l in range(L):
                    tot = tot + plsc.load_gather(pr_v, [flat + l])
                sc_v[pl.ds(q * L, L)] = tot
            base = pl.multiple_of(w * EPW + c * K, 8)
            pltpu.sync_copy(sc_v, out_hbm.at[pl.ds(base, K)])

            @pl.when(g < NGRP - 1)
            def _refill():
                adesc(c + NB, b).start()
                bdesc(c + NB, b).start()

        return _

    lax.fori_loop(0, NGRP, gbody, None)


@functools.lru_cache(maxsize=None)
def _sc_calls():
    mesh = plsc.VectorSubcoreMesh(core_axis_name="c", subcore_axis_name="s",
                                  num_cores=NC, num_subcores=NS)
    params = pltpu.CompilerParams(needs_layout_passes=False)
    deg_call = functools.partial(
        pl.kernel,
        out_type=jax.ShapeDtypeStruct((NC, N_PAD, L), jnp.float32),
        mesh=mesh,
        scratch_types=[
            pltpu.VMEM((NCHUNK, K), jnp.int32),
            pltpu.VMEM((K,), jnp.float32),
            pltpu.VMEM((NPT,), jnp.float32),
            pltpu.VMEM((NPT, L), jnp.float32),
            pltpu.SemaphoreType.DMA,
            pltpu.VMEM_SHARED((N_PAD,), jnp.float32),
        ],
        compiler_params=params,
    )(_deg_body)
    spass_call = functools.partial(
        pl.kernel,
        out_type=jax.ShapeDtypeStruct((NC, N_PAD, D), jnp.float32),
        mesh=mesh,
        scratch_types=[
            pltpu.VMEM((NB, K2), jnp.int32),
            pltpu.VMEM((NB, K2), jnp.int32),
            pltpu.VMEM((NB * K2, D), jnp.float32),
        ] + [pltpu.SemaphoreType.DMA] * (4 * NB) + [
            pltpu.VMEM_SHARED((N_PAD, D), jnp.float32),
        ],
        compiler_params=params,
    )(_spass_body)
    score_call = functools.partial(
        pl.kernel,
        out_type=jax.ShapeDtypeStruct((N_EDGES,), jnp.float32),
        mesh=mesh,
        name="edge_score",
        scratch_types=[
            pltpu.VMEM((EPW,), jnp.int32),
            pltpu.VMEM((EPW,), jnp.int32),
            pltpu.VMEM((NB, K, D), jnp.float32),
            pltpu.VMEM((NB, K, D), jnp.float32),
            pltpu.VMEM((K * L,), jnp.float32),
            pltpu.VMEM((K,), jnp.float32),
        ] + [pltpu.SemaphoreType.DMA] * NB,
        compiler_params=params,
    )(_score_body)
    return deg_call, spass_call, score_call


def _dinv_of(degrep_ref):
    deg = (degrep_ref[0, 0:N_NODES, 0:1] + degrep_ref[1, 0:N_NODES, 0:1]
           + 1.0)
    return lax.rsqrt(deg)


def _tc1_body(x_ref, w_ref, degrep_ref, o_ref):
    dinv = _dinv_of(degrep_ref)
    xw = jnp.dot(x_ref[:], w_ref[:], preferred_element_type=jnp.float32)
    o_ref[:] = xw * dinv


def _tc2_body(s1p_ref, xs1_ref, w_ref, b_ref, degrep_ref, o_ref):
    dinv = _dinv_of(degrep_ref)
    h = s1p_ref[0, 0:N_NODES, :] + s1p_ref[1, 0:N_NODES, :] + xs1_ref[:]
    h = jnp.maximum(b_ref[:][None, :] + dinv * h, 0.0)
    hw = jnp.dot(h, w_ref[:], preferred_element_type=jnp.float32)
    o_ref[:] = hw * dinv


def _tc3_body(s2p_ref, xs2_ref, b_ref, degrep_ref, o_ref):
    dinv = _dinv_of(degrep_ref)
    o_ref[:] = b_ref[:][None, :] + dinv * (s2p_ref[0, 0:N_NODES, :]
                                           + s2p_ref[1, 0:N_NODES, :]
                                           + xs2_ref[:])


_tc1_call = pl.pallas_call(
    _tc1_body, out_shape=jax.ShapeDtypeStruct((N_NODES, D), jnp.float32))
_tc2_call = pl.pallas_call(
    _tc2_body, out_shape=jax.ShapeDtypeStruct((N_NODES, D), jnp.float32))
_tc3_call = pl.pallas_call(
    _tc3_body, out_shape=jax.ShapeDtypeStruct((N_NODES, D), jnp.float32))


def kernel(x, edge_index, W1, b1, W2, b2):
    deg_call, spass_call, score_call = _sc_calls()
    src = edge_index[0].astype(jnp.int32)
    dst = edge_index[1].astype(jnp.int32)
    dst3 = dst.reshape(NW, NCHUNK, K)
    degrep = deg_call(dst3)
    xs1 = _tc1_call(x, W1, degrep)
    s1p = spass_call(xs1, src, dst)
    xs2 = _tc2_call(s1p, xs1, W2, b1, degrep)
    s2p = spass_call(xs2, src, dst)
    h2 = _tc3_call(s2p, xs2, b2, degrep)
    score = score_call(h2, src, dst)
    return score


# speedup vs baseline: 1.0522x; 1.0522x over previous
import functools

import jax
import jax.numpy as jnp
from jax import lax
from jax.experimental import pallas as pl
from jax.experimental.pallas import tpu as pltpu
from jax.experimental.pallas import tpu_sc as plsc

N_NODES = 10000
N_PAD = 10240
N_EDGES = 320000
D = 128
NC = 2
NS = 16
L = 16
NW = NC * NS
EPW = N_EDGES // NW
K = 80
NCHUNK = EPW // K
NB = 5
NGRP = NCHUNK // NB
NPT = N_PAD // NS
RPT = N_PAD // NS
K2 = 40
NCHUNK2 = EPW // K2
NGRP2 = NCHUNK2 // NB


def _worker(cid, sid):
    return cid * NS + sid


def _deg_body(dst3_hbm, out_hbm, didx_v, ones_v, stage_v, rep_v, dsem,
              acc_sh):
    cid = lax.axis_index("c")
    sid = lax.axis_index("s")
    w = _worker(cid, sid)

    for i in range(K // L):
        ones_v[pl.ds(i * L, L)] = jnp.ones((L,), jnp.float32)

    def zbody(i, _):
        stage_v[pl.ds(i * L, L)] = jnp.zeros((L,), jnp.float32)
        return _

    lax.fori_loop(0, NPT // L, zbody, None)
    pltpu.sync_copy(stage_v, acc_sh.at[pl.ds(sid * NPT, NPT)])
    pltpu.sync_copy(dst3_hbm.at[w], didx_v)
    plsc.subcore_barrier()

    def sdesc(c):
        return pltpu.make_async_copy(ones_v, acc_sh.at[didx_v.at[c]], dsem)

    def gbody(g, _):
        for b in range(NB):
            sdesc(g * NB + b).start(add=True)
        for b in range(NB):
            sdesc(g * NB + b).wait()
        return _

    lax.fori_loop(0, NGRP, gbody, None)
    plsc.subcore_barrier()

    pltpu.sync_copy(acc_sh.at[pl.ds(sid * NPT, NPT)], stage_v)

    def rbody(i, _):
        rep_v[i, :] = plsc.load_gather(stage_v, [jnp.full((L,), i, jnp.int32)])
        return _

    lax.fori_loop(0, NPT, rbody, None)
    pltpu.sync_copy(rep_v, out_hbm.at[cid, pl.ds(sid * NPT, NPT), :])


def _spass_body(xs_hbm, src_hbm, dst_hbm, out_hbm,
                sidx_v, didx_v, rows_v,
                gs0, gs1, gs2, gs3, gs4, ss0, ss1, ss2, ss3, ss4,
                is0, is1, is2, is3, is4, js0, js1, js2, js3, js4, acc_sh):
    cid = lax.axis_index("c")
    sid = lax.axis_index("s")
    w = _worker(cid, sid)
    gsems = [gs0, gs1, gs2, gs3, gs4]
    ssems = [ss0, ss1, ss2, ss3, ss4]
    isems = [is0, is1, is2, is3, is4]
    jsems = [js0, js1, js2, js3, js4]

    def zbody(i, _):
        for j in range(D // L):
            rows_v[i, pl.ds(j * L, L)] = jnp.zeros((L,), jnp.float32)
        return _

    lax.fori_loop(0, NB * K2, zbody, None)
    zrows = NB * K2
    done = 0
    while done < RPT:
        n = min(zrows, RPT - done)
        pltpu.sync_copy(rows_v.at[pl.ds(0, n)],
                        acc_sh.at[pl.ds(sid * RPT + done, n), :])
        done += n

    def sidesc(c, b):
        base = pl.multiple_of(w * EPW + c * K2, 8)
        return pltpu.make_async_copy(src_hbm.at[pl.ds(base, K2)],
                                     sidx_v.at[b], isems[b])

    def didesc(c, b):
        base = pl.multiple_of(w * EPW + c * K2, 8)
        return pltpu.make_async_copy(dst_hbm.at[pl.ds(base, K2)],
                                     didx_v.at[b], jsems[b])

    def gdesc(c, b):
        del c
        return pltpu.make_async_copy(
            xs_hbm.at[sidx_v.at[b]], rows_v.at[pl.ds(b * K2, K2)], gsems[b])

    def sdesc(c, b):
        del c
        return pltpu.make_async_copy(
            rows_v.at[pl.ds(b * K2, K2)], acc_sh.at[didx_v.at[b]], ssems[b])

    for b in range(NB):
        sidesc(b, b).start()
        sidesc(b, b).wait()
        gdesc(b, b).start()
        didesc(b, b).start()
    plsc.subcore_barrier()

    def gbody(g, _):
        for b in range(NB):
            c = g * NB + b
            gdesc(c, b).wait()
            didesc(c, b).wait()
            sdesc(c, b).start(add=True)

            @pl.when(g < NGRP2 - 1)
            def _stage_sidx():
                sidesc(c + NB, b).start()

        @pl.when(g < NGRP2 - 1)
        def _refill():
            for b in range(NB):
                c = g * NB + b
                sdesc(c, b).wait()
                didesc(c + NB, b).start()
                sidesc(c + NB, b).wait()
                gdesc(c + NB, b).start()

        return _

    lax.fori_loop(0, NGRP2, gbody, None)
    for b in range(NB):
        sdesc((NGRP2 - 1) * NB + b, b).wait()
    plsc.subcore_barrier()

    pltpu.sync_copy(acc_sh.at[pl.ds(sid * RPT, RPT), :],
                    out_hbm.at[cid, pl.ds(sid * RPT, RPT), :])


def _score_body(h_hbm, src_hbm, dst_hbm, out_hbm,
                sidx_v, didx_v, hs_v, hd_v, pr_v, sc_v,
                s0, s1, s2, s3, s4):
    cid = lax.axis_index("c")
    sid = lax.axis_index("s")
    w = _worker(cid, sid)
    sems = [s0, s1, s2, s3, s4]
    iota = lax.iota(jnp.int32, L)

    pltpu.sync_copy(src_hbm.at[pl.ds(w * EPW, EPW)], sidx_v)
    pltpu.sync_copy(dst_hbm.at[pl.ds(w * EPW, EPW)], didx_v)

    def adesc(c, b):
        off = pl.multiple_of(c * K, 8)
        return pltpu.make_async_copy(
            h_hbm.at[sidx_v.at[pl.ds(off, K)]], hs_v.at[b], sems[b])

    def bdesc(c, b):
        off = pl.multiple_of(c * K, 8)
        return pltpu.make_async_copy(
            h_hbm.at[didx_v.at[pl.ds(off, K)]], hd_v.at[b], sems[b])

    for b in range(NB):
        adesc(b, b).start()
        bdesc(b, b).start()

    def gbody(g, _):
        for b in range(NB):
            c = g * NB + b
            adesc(c, b).wait()
            bdesc(c, b).wait()

            @plsc.parallel_loop(0, K, unroll=4)
            def _edot(e):
                acc = jnp.zeros((L,), jnp.float32)
                for k in range(D // L):
                    acc = acc + (hs_v[b, e, pl.ds(k * L, L)] *
                                 hd_v[b, e, pl.ds(k * L, L)])
                pr_v[pl.ds(pl.multiple_of(e * L, 8), L)] = acc

            for q in range(K // L):
                flat = (q * L + iota) * L
                tot = jnp.zeros((L,), jnp.float32)
                for l in range(L):
                    tot = tot + plsc.load_gather(pr_v, [flat + l])
                sc_v[pl.ds(q * L, L)] = tot
            base = pl.multiple_of(w * EPW + c * K, 8)
            pltpu.sync_copy(sc_v, out_hbm.at[pl.ds(base, K)])

            @pl.when(g < NGRP - 1)
            def _refill():
                adesc(c + NB, b).start()
                bdesc(c + NB, b).start()

        return _

    lax.fori_loop(0, NGRP, gbody, None)


@functools.lru_cache(maxsize=None)
def _sc_calls():
    mesh = plsc.VectorSubcoreMesh(core_axis_name="c", subcore_axis_name="s",
                                  num_cores=NC, num_subcores=NS)
    params = pltpu.CompilerParams(needs_layout_passes=False)
    deg_call = functools.partial(
        pl.kernel,
        out_type=jax.ShapeDtypeStruct((NC, N_PAD, L), jnp.float32),
        mesh=mesh,
        scratch_types=[
            pltpu.VMEM((NCHUNK, K), jnp.int32),
            pltpu.VMEM((K,), jnp.float32),
            pltpu.VMEM((NPT,), jnp.float32),
            pltpu.VMEM((NPT, L), jnp.float32),
            pltpu.SemaphoreType.DMA,
            pltpu.VMEM_SHARED((N_PAD,), jnp.float32),
        ],
        compiler_params=params,
    )(_deg_body)
    spass_call = functools.partial(
        pl.kernel,
        out_type=jax.ShapeDtypeStruct((NC, N_PAD, D), jnp.float32),
        mesh=mesh,
        scratch_types=[
            pltpu.VMEM((NB, K2), jnp.int32),
            pltpu.VMEM((NB, K2), jnp.int32),
            pltpu.VMEM((NB * K2, D), jnp.float32),
        ] + [pltpu.SemaphoreType.DMA] * (4 * NB) + [
            pltpu.VMEM_SHARED((N_PAD, D), jnp.float32),
        ],
        compiler_params=params,
    )(_spass_body)
    score_call = functools.partial(
        pl.kernel,
        out_type=jax.ShapeDtypeStruct((N_EDGES,), jnp.float32),
        mesh=mesh,
        name="edge_score",
        scratch_types=[
            pltpu.VMEM((EPW,), jnp.int32),
            pltpu.VMEM((EPW,), jnp.int32),
            pltpu.VMEM((NB, K, D), jnp.float32),
            pltpu.VMEM((NB, K, D), jnp.float32),
            pltpu.VMEM((K * L,), jnp.float32),
            pltpu.VMEM((K,), jnp.float32),
        ] + [pltpu.SemaphoreType.DMA] * NB,
        compiler_params=params,
    )(_score_body)
    return deg_call, spass_call, score_call


def _dinv_of(degrep_ref):
    deg = (degrep_ref[0, 0:N_NODES, 0:1] + degrep_ref[1, 0:N_NODES, 0:1]
           + 1.0)
    return lax.rsqrt(deg)


def _tc1_body(x_ref, w_ref, degrep_ref, o_ref):
    dinv = _dinv_of(degrep_ref)
    xw = jnp.dot(x_ref[:], w_ref[:], preferred_element_type=jnp.float32)
    o_ref[:] = xw * dinv


def _tc2_body(s1p_ref, xs1_ref, w_ref, b_ref, degrep_ref, o_ref):
    dinv = _dinv_of(degrep_ref)
    h = s1p_ref[0, 0:N_NODES, :] + s1p_ref[1, 0:N_NODES, :] + xs1_ref[:]
    h = jnp.maximum(b_ref[:][None, :] + dinv * h, 0.0)
    hw = jnp.dot(h, w_ref[:], preferred_element_type=jnp.float32)
    o_ref[:] = hw * dinv


def _tc3_body(s2p_ref, xs2_ref, b_ref, degrep_ref, o_ref):
    dinv = _dinv_of(degrep_ref)
    o_ref[:] = b_ref[:][None, :] + dinv * (s2p_ref[0, 0:N_NODES, :]
                                           + s2p_ref[1, 0:N_NODES, :]
                                           + xs2_ref[:])


_tc1_call = pl.pallas_call(
    _tc1_body, out_shape=jax.ShapeDtypeStruct((N_NODES, D), jnp.float32))
_tc2_call = pl.pallas_call(
    _tc2_body, out_shape=jax.ShapeDtypeStruct((N_NODES, D), jnp.float32))
_tc3_call = pl.pallas_call(
    _tc3_body, out_shape=jax.ShapeDtypeStruct((N_NODES, D), jnp.float32))


def kernel(x, edge_index, W1, b1, W2, b2):
    deg_call, spass_call, score_call = _sc_calls()
    src = edge_index[0].astype(jnp.int32)
    dst = edge_index[1].astype(jnp.int32)
    dst3 = dst.reshape(NW, NCHUNK, K)
    degrep = deg_call(dst3)
    xs1 = _tc1_call(x, W1, degrep)
    s1p = spass_call(xs1, src, dst)
    xs2 = _tc2_call(s1p, xs1, W2, b1, degrep)
    s2p = spass_call(xs2, src, dst)
    h2 = _tc3_call(s2p, xs2, b2, degrep)
    score = score_call(h2, src, dst)
    return score


# speedup vs baseline: 1.0681x; 1.0151x over previous
import functools

import jax
import jax.numpy as jnp
from jax import lax
from jax.experimental import pallas as pl
from jax.experimental.pallas import tpu as pltpu
from jax.experimental.pallas import tpu_sc as plsc

N_NODES = 10000
N_PAD = 10240
N_EDGES = 320000
D = 128
NC = 2
NS = 16
L = 16
NW = NC * NS
EPW = N_EDGES // NW
K = 80
NCHUNK = EPW // K
NB = 5
NGRP = NCHUNK // NB
NPT = N_PAD // NS
RPT = N_PAD // NS
K2 = 40
NCHUNK2 = EPW // K2
NGRP2 = NCHUNK2 // NB


def _worker(cid, sid):
    return cid * NS + sid


def _deg_body(dst3_hbm, out_hbm, didx_v, ones_v, stage_v, rep_v,
              ds0, ds1, ds2, ds3, ds4, acc_sh):
    cid = lax.axis_index("c")
    sid = lax.axis_index("s")
    w = _worker(cid, sid)
    dsems = [ds0, ds1, ds2, ds3, ds4]

    for i in range(K // L):
        ones_v[pl.ds(i * L, L)] = jnp.ones((L,), jnp.float32)

    def zbody(i, _):
        stage_v[pl.ds(i * L, L)] = jnp.zeros((L,), jnp.float32)
        return _

    lax.fori_loop(0, NPT // L, zbody, None)
    pltpu.sync_copy(stage_v, acc_sh.at[pl.ds(sid * NPT, NPT)])
    pltpu.sync_copy(dst3_hbm.at[w], didx_v)
    plsc.subcore_barrier()

    def sdesc(c, b):
        return pltpu.make_async_copy(ones_v, acc_sh.at[didx_v.at[c]],
                                     dsems[b])

    for b in range(NB):
        sdesc(b, b).start(add=True)

    def gbody(g, _):
        for b in range(NB):
            c = g * NB + b
            sdesc(c, b).wait()

            @pl.when(g < NGRP - 1)
            def _next():
                sdesc(c + NB, b).start(add=True)

        return _

    lax.fori_loop(0, NGRP, gbody, None)
    plsc.subcore_barrier()

    pltpu.sync_copy(acc_sh.at[pl.ds(sid * NPT, NPT)], stage_v)

    @plsc.parallel_loop(0, NPT, unroll=4)
    def _rep(i):
        rep_v[i, :] = plsc.load_gather(stage_v, [jnp.full((L,), i, jnp.int32)])

    pltpu.sync_copy(rep_v, out_hbm.at[cid, pl.ds(sid * NPT, NPT), :])


def _spass_body(xs_hbm, src_hbm, dst_hbm, out_hbm,
                sidx_v, didx_v, rows_v,
                gs0, gs1, gs2, gs3, gs4, ss0, ss1, ss2, ss3, ss4,
                is0, is1, is2, is3, is4, js0, js1, js2, js3, js4, acc_sh):
    cid = lax.axis_index("c")
    sid = lax.axis_index("s")
    w = _worker(cid, sid)
    gsems = [gs0, gs1, gs2, gs3, gs4]
    ssems = [ss0, ss1, ss2, ss3, ss4]
    isems = [is0, is1, is2, is3, is4]
    jsems = [js0, js1, js2, js3, js4]

    def zbody(i, _):
        for j in range(D // L):
            rows_v[i, pl.ds(j * L, L)] = jnp.zeros((L,), jnp.float32)
        return _

    lax.fori_loop(0, NB * K2, zbody, None)
    zrows = NB * K2
    zplan = []
    done = 0
    while done < RPT:
        n = min(zrows, RPT - done)
        zplan.append((done, n))
        done += n

    def zdesc(i):
        off, n = zplan[i]
        return pltpu.make_async_copy(
            rows_v.at[pl.ds(0, n)],
            acc_sh.at[pl.ds(sid * RPT + off, n), :], ssems[i])

    for i in range(len(zplan)):
        zdesc(i).start()

    def sidesc(c, b):
        base = pl.multiple_of(w * EPW + c * K2, 8)
        return pltpu.make_async_copy(src_hbm.at[pl.ds(base, K2)],
                                     sidx_v.at[b], isems[b])

    def didesc(c, b):
        base = pl.multiple_of(w * EPW + c * K2, 8)
        return pltpu.make_async_copy(dst_hbm.at[pl.ds(base, K2)],
                                     didx_v.at[b], jsems[b])

    def gdesc(c, b):
        del c
        return pltpu.make_async_copy(
            xs_hbm.at[sidx_v.at[b]], rows_v.at[pl.ds(b * K2, K2)], gsems[b])

    def sdesc(c, b):
        del c
        return pltpu.make_async_copy(
            rows_v.at[pl.ds(b * K2, K2)], acc_sh.at[didx_v.at[b]], ssems[b])

    for b in range(NB):
        sidesc(b, b).start()
    for b in range(NB):
        sidesc(b, b).wait()
    for i in range(len(zplan)):
        zdesc(i).wait()
    for b in range(NB):
        gdesc(b, b).start()
        didesc(b, b).start()
    plsc.subcore_barrier()

    def gbody(g, _):
        for b in range(NB):
            c = g * NB + b
            gdesc(c, b).wait()
            didesc(c, b).wait()
            sdesc(c, b).start(add=True)

            @pl.when(g < NGRP2 - 1)
            def _stage_sidx():
                sidesc(c + NB, b).start()

        @pl.when(g < NGRP2 - 1)
        def _refill():
            for b in range(NB):
                c = g * NB + b
                sdesc(c, b).wait()
                didesc(c + NB, b).start()
                sidesc(c + NB, b).wait()
                gdesc(c + NB, b).start()

        return _

    lax.fori_loop(0, NGRP2, gbody, None)
    for b in range(NB):
        sdesc((NGRP2 - 1) * NB + b, b).wait()
    plsc.subcore_barrier()

    pltpu.sync_copy(acc_sh.at[pl.ds(sid * RPT, RPT), :],
                    out_hbm.at[cid, pl.ds(sid * RPT, RPT), :])


def _score_body(h_hbm, src_hbm, dst_hbm, out_hbm,
                sidx_v, didx_v, hs_v, hd_v, pr_v, sc_v,
                s0, s1, s2, s3, s4):
    cid = lax.axis_index("c")
    sid = lax.axis_index("s")
    w = _worker(cid, sid)
    sems = [s0, s1, s2, s3, s4]
    iota = lax.iota(jnp.int32, L)

    pltpu.sync_copy(src_hbm.at[pl.ds(w * EPW, EPW)], sidx_v)
    pltpu.sync_copy(dst_hbm.at[pl.ds(w * EPW, EPW)], didx_v)

    def adesc(c, b):
        off = pl.multiple_of(c * K, 8)
        return pltpu.make_async_copy(
            h_hbm.at[sidx_v.at[pl.ds(off, K)]], hs_v.at[b], sems[b])

    def bdesc(c, b):
        off = pl.multiple_of(c * K, 8)
        return pltpu.make_async_copy(
            h_hbm.at[didx_v.at[pl.ds(off, K)]], hd_v.at[b], sems[b])

    for b in range(NB):
        adesc(b, b).start()
        bdesc(b, b).start()

    def gbody(g, _):
        for b in range(NB):
            c = g * NB + b
            adesc(c, b).wait()
            bdesc(c, b).wait()

            @plsc.parallel_loop(0, K, unroll=4)
            def _edot(e):
                acc = jnp.zeros((L,), jnp.float32)
                for k in range(D // L):
                    acc = acc + (hs_v[b, e, pl.ds(k * L, L)] *
                                 hd_v[b, e, pl.ds(k * L, L)])
                pr_v[pl.ds(pl.multiple_of(e * L, 8), L)] = acc

            for q in range(K // L):
                flat = (q * L + iota) * L
                tot = jnp.zeros((L,), jnp.float32)
                for l in range(L):
                    tot = tot + plsc.load_gather(pr_v, [flat + l])
                sc_v[pl.ds(q * L, L)] = tot
            base = pl.multiple_of(w * EPW + c * K, 8)
            pltpu.sync_copy(sc_v, out_hbm.at[pl.ds(base, K)])

            @pl.when(g < NGRP - 1)
            def _refill():
                adesc(c + NB, b).start()
                bdesc(c + NB, b).start()

        return _

    lax.fori_loop(0, NGRP, gbody, None)


@functools.lru_cache(maxsize=None)
def _sc_calls():
    mesh = plsc.VectorSubcoreMesh(core_axis_name="c", subcore_axis_name="s",
                                  num_cores=NC, num_subcores=NS)
    params = pltpu.CompilerParams(needs_layout_passes=False)
    deg_call = functools.partial(
        pl.kernel,
        out_type=jax.ShapeDtypeStruct((NC, N_PAD, L), jnp.float32),
        mesh=mesh,
        scratch_types=[
            pltpu.VMEM((NCHUNK, K), jnp.int32),
            pltpu.VMEM((K,), jnp.float32),
            pltpu.VMEM((NPT,), jnp.float32),
            pltpu.VMEM((NPT, L), jnp.float32),
        ] + [pltpu.SemaphoreType.DMA] * NB + [
            pltpu.VMEM_SHARED((N_PAD,), jnp.float32),
        ],
        compiler_params=params,
    )(_deg_body)
    spass_call = functools.partial(
        pl.kernel,
        out_type=jax.ShapeDtypeStruct((NC, N_PAD, D), jnp.float32),
        mesh=mesh,
        scratch_types=[
            pltpu.VMEM((NB, K2), jnp.int32),
            pltpu.VMEM((NB, K2), jnp.int32),
            pltpu.VMEM((NB * K2, D), jnp.float32),
        ] + [pltpu.SemaphoreType.DMA] * (4 * NB) + [
            pltpu.VMEM_SHARED((N_PAD, D), jnp.float32),
        ],
        compiler_params=params,
    )(_spass_body)
    score_call = functools.partial(
        pl.kernel,
        out_type=jax.ShapeDtypeStruct((N_EDGES,), jnp.float32),
        mesh=mesh,
        name="edge_score",
        scratch_types=[
            pltpu.VMEM((EPW,), jnp.int32),
            pltpu.VMEM((EPW,), jnp.int32),
            pltpu.VMEM((NB, K, D), jnp.float32),
            pltpu.VMEM((NB, K, D), jnp.float32),
            pltpu.VMEM((K * L,), jnp.float32),
            pltpu.VMEM((K,), jnp.float32),
        ] + [pltpu.SemaphoreType.DMA] * NB,
        compiler_params=params,
    )(_score_body)
    return deg_call, spass_call, score_call


def _dinv_of(degrep_ref):
    deg = (degrep_ref[0, 0:N_NODES, 0:1] + degrep_ref[1, 0:N_NODES, 0:1]
           + 1.0)
    return lax.rsqrt(deg)


def _tc1_body(x_ref, w_ref, degrep_ref, o_ref):
    dinv = _dinv_of(degrep_ref)
    xw = jnp.dot(x_ref[:], w_ref[:], preferred_element_type=jnp.float32)
    o_ref[:] = xw * dinv


def _tc2_body(s1p_ref, xs1_ref, w_ref, b_ref, degrep_ref, o_ref):
    dinv = _dinv_of(degrep_ref)
    h = s1p_ref[0, 0:N_NODES, :] + s1p_ref[1, 0:N_NODES, :] + xs1_ref[:]
    h = jnp.maximum(b_ref[:][None, :] + dinv * h, 0.0)
    hw = jnp.dot(h, w_ref[:], preferred_element_type=jnp.float32)
    o_ref[:] = hw * dinv


def _tc3_body(s2p_ref, xs2_ref, b_ref, degrep_ref, o_ref):
    dinv = _dinv_of(degrep_ref)
    o_ref[:] = b_ref[:][None, :] + dinv * (s2p_ref[0, 0:N_NODES, :]
                                           + s2p_ref[1, 0:N_NODES, :]
                                           + xs2_ref[:])


_tc1_call = pl.pallas_call(
    _tc1_body, out_shape=jax.ShapeDtypeStruct((N_NODES, D), jnp.float32))
_tc2_call = pl.pallas_call(
    _tc2_body, out_shape=jax.ShapeDtypeStruct((N_NODES, D), jnp.float32))
_tc3_call = pl.pallas_call(
    _tc3_body, out_shape=jax.ShapeDtypeStruct((N_NODES, D), jnp.float32))


def kernel(x, edge_index, W1, b1, W2, b2):
    deg_call, spass_call, score_call = _sc_calls()
    src = edge_index[0].astype(jnp.int32)
    dst = edge_index[1].astype(jnp.int32)
    dst3 = dst.reshape(NW, NCHUNK, K)
    degrep = deg_call(dst3)
    xs1 = _tc1_call(x, W1, degrep)
    s1p = spass_call(xs1, src, dst)
    xs2 = _tc2_call(s1p, xs1, W2, b1, degrep)
    s2p = spass_call(xs2, src, dst)
    h2 = _tc3_call(s2p, xs2, b2, degrep)
    score = score_call(h2, src, dst)
    return score
